# baseline (speedup 1.0000x reference)
import functools

import jax
import jax.numpy as jnp
from jax import lax
from jax.experimental import pallas as pl
from jax.experimental.pallas import tpu as pltpu
from jax.experimental.pallas import tpu_sc as plsc

NC = 2
NS = 16
L = 16
NW = NC * NS


def _make_sc_kernel(B, K, D, V):
    BW = B // NW
    C = 16
    H = 4
    NCH = BW // C
    Q = D // L

    mesh = plsc.VectorSubcoreMesh(core_axis_name="c", subcore_axis_name="s")

    @functools.partial(
        pl.kernel,
        out_type=jax.ShapeDtypeStruct((B * 2 * L,), jnp.float32),
        mesh=mesh,
        scratch_types=[
            pltpu.VMEM((BW,), jnp.int32),
            pltpu.VMEM((BW,), jnp.int32),
            pltpu.VMEM((BW * K + 2 * L,), jnp.int32),
            pltpu.VMEM((H * 8, D), jnp.float32),
            pltpu.VMEM((H * 8, D), jnp.float32),
            pltpu.VMEM((H * K * 8, D), jnp.float32),
            pltpu.VMEM((C * 2 * L,), jnp.float32),
            pltpu.SemaphoreType.DMA,
        ],
        compiler_params=pltpu.CompilerParams(needs_layout_passes=False),
    )
    def sg_kernel(tw_hbm, cw_hbm, neg_hbm, tt_hbm, ct_hbm, out_hbm,
                  idx_t, idx_c, idx_n, grp_t, grp_c, grp_n, acc, sem):
        wid = lax.axis_index("s") * NC + lax.axis_index("c")
        base_w = wid * BW

        pltpu.sync_copy(tw_hbm.at[pl.ds(base_w, BW)], idx_t)
        pltpu.sync_copy(cw_hbm.at[pl.ds(base_w, BW)], idx_c)
        pltpu.sync_copy(neg_hbm.at[pl.ds(base_w * K, BW * K)],
                        idx_n.at[pl.ds(0, BW * K)])

        lane = lax.iota(jnp.int32, L)

        def chunk_body(ci, carry):
            base = ci * C
            tvec = idx_t[pl.ds(base, L)]
            cvec = idx_c[pl.ds(base, L)]
            ta = (tvec >> 3) << 3
            ca = (cvec >> 3) << 3

            for h in range(C // H):
                copies = []
                nsub = []
                for e8 in range(H):
                    e = h * H + e8
                    copies.append(pltpu.async_copy(
                        tt_hbm.at[pl.ds(pl.multiple_of(ta[e], 8), 8)],
                        grp_t.at[pl.ds(e8 * 8, 8)], sem))
                    copies.append(pltpu.async_copy(
                        ct_hbm.at[pl.ds(pl.multiple_of(ca[e], 8), 8)],
                        grp_c.at[pl.ds(e8 * 8, 8)], sem))
                    el = base + e
                    nv0 = idx_n[pl.ds(el * K, L)]
                    nv1 = idx_n[pl.ds(el * K + L, L)]
                    na0 = (nv0 >> 3) << 3
                    na1 = (nv1 >> 3) << 3
                    ns0 = nv0 & 7
                    ns1 = nv1 & 7
                    nsub.append((ns0, ns1))
                    for k in range(K):
                        a = na0[k] if k < L else na1[k - L]
                        copies.append(pltpu.async_copy(
                            ct_hbm.at[pl.ds(pl.multiple_of(a, 8), 8)],
                            grp_n.at[pl.ds((e8 * K + k) * 8, 8)], sem))
                for cp in copies:
                    cp.wait()

                for e8 in range(H):
                    e = h * H + e8
                    st = tvec[e] & 7
                    sc = cvec[e] & 7
                    t = [grp_t[e8 * 8 + st, pl.ds(q * L, L)] for q in range(Q)]
                    c = [grp_c[e8 * 8 + sc, pl.ds(q * L, L)] for q in range(Q)]
                    p = t[0] * c[0]
                    for q in range(1, Q):
                        p = p + t[q] * c[q]
                    v0 = jnp.where(lane == 0, jnp.sum(p), 0.0)
                    v1 = jnp.zeros((L,), jnp.float32)
                    ns0, ns1 = nsub[e8]
                    for k in range(K):
                        sn = ns0[k] if k < L else ns1[k - L]
                        gr = (e8 * K + k) * 8 + sn
                        s = t[0] * grp_n[gr, pl.ds(0, L)]
                        for q in range(1, Q):
                            s = s + t[q] * grp_n[gr, pl.ds(q * L, L)]
                        col = 1 + k
                        if col < L:
                            v0 = jnp.where(lane == col, jnp.sum(s), v0)
                        else:
                            v1 = jnp.where(lane == col - L, jnp.sum(s), v1)
                    acc[pl.ds(e * 2 * L, L)] = v0
                    acc[pl.ds(e * 2 * L + L, L)] = v1

            pltpu.sync_copy(
                acc, out_hbm.at[pl.ds((base_w + base) * 2 * L, C * 2 * L)])
            return carry

        lax.fori_loop(0, NCH, chunk_body, 0)

    return sg_kernel


def kernel(target_word, context_word, negative_samples, target_table, context_table):
    B = target_word.shape[0]
    K = negative_samples.shape[1]
    V, D = target_table.shape
    tw = target_word.astype(jnp.int32)
    cw = context_word.astype(jnp.int32)
    neg = negative_samples.astype(jnp.int32).reshape(B * K)
    sg = _make_sc_kernel(B, K, D, V)
    out = sg(tw, cw, neg, target_table, context_table)
    return out.reshape(B, 2 * L)[:, :1 + K]


# speedup vs baseline: 1.0270x; 1.0270x over previous
import functools

import jax
import jax.numpy as jnp
from jax import lax
from jax.experimental import pallas as pl
from jax.experimental.pallas import tpu as pltpu
from jax.experimental.pallas import tpu_sc as plsc

NC = 2
NS = 16
L = 16
NW = NC * NS
W = 128


def _make_sc_kernel(B, K, D, V):
    BW = B // NW
    C = 16
    NCH = BW // C
    Q = D // L
    CK = C * K
    NSTR = -(-CK // 128)

    mesh = plsc.VectorSubcoreMesh(core_axis_name="c", subcore_axis_name="s")

    @functools.partial(
        pl.kernel,
        out_type=jax.ShapeDtypeStruct((B * 2 * L,), jnp.float32),
        mesh=mesh,
        scratch_types=[
            pltpu.VMEM((BW,), jnp.int32),
            pltpu.VMEM((BW,), jnp.int32),
            pltpu.VMEM((BW * K + 2 * L,), jnp.int32),
            pltpu.VMEM((C,), jnp.int32),
            pltpu.VMEM((C,), jnp.int32),
            pltpu.VMEM((CK + 4 * L,), jnp.int32),
            pltpu.VMEM((C, W), jnp.float32),
            pltpu.VMEM((C, W), jnp.float32),
            pltpu.VMEM((CK, W), jnp.float32),
            pltpu.VMEM((C * 2 * L,), jnp.float32),
            pltpu.SemaphoreType.DMA,
        ],
        compiler_params=pltpu.CompilerParams(needs_layout_passes=False),
    )
    def sg_kernel(tw_hbm, cw_hbm, neg_hbm, tt_hbm, ct_hbm, out_hbm,
                  idx_t, idx_c, idx_n, hid_t, hid_c, hid_n,
                  rows_t, rows_c, rows_n, acc, sem):
        wid = lax.axis_index("s") * NC + lax.axis_index("c")
        base_w = wid * BW

        pltpu.sync_copy(tw_hbm.at[pl.ds(base_w, BW)], idx_t)
        pltpu.sync_copy(cw_hbm.at[pl.ds(base_w, BW)], idx_c)
        pltpu.sync_copy(neg_hbm.at[pl.ds(base_w * K, BW * K)],
                        idx_n.at[pl.ds(0, BW * K)])

        lane = lax.iota(jnp.int32, L)

        def chunk_body(ci, carry):
            base = ci * C
            tvec = idx_t[pl.ds(base, L)]
            cvec = idx_c[pl.ds(base, L)]
            hid_t[pl.ds(0, L)] = tvec >> 1
            hid_c[pl.ds(0, L)] = cvec >> 1
            nvecs = []
            for i in range(CK // L):
                nv = idx_n[pl.ds(base * K + i * L, L)]
                hid_n[pl.ds(i * L, L)] = nv >> 1
                nvecs.append(nv)

            copies = [
                pltpu.async_copy(tt_hbm.at[hid_t], rows_t, sem),
                pltpu.async_copy(ct_hbm.at[hid_c], rows_c, sem),
            ]
            for j in range(NSTR):
                n = min(128, CK - j * 128)
                copies.append(
                    pltpu.async_copy(ct_hbm.at[hid_n.at[pl.ds(j * 128, n)]],
                                     rows_n.at[pl.ds(j * 128, n)], sem))
            for cp in copies:
                cp.wait()

            for e in range(C):
                pt = (tvec[e] & 1) * D
                pc = (cvec[e] & 1) * D
                t = [rows_t[e, pl.ds(pt + q * L, L)] for q in range(Q)]
                c = [rows_c[e, pl.ds(pc + q * L, L)] for q in range(Q)]
                p = t[0] * c[0]
                for q in range(1, Q):
                    p = p + t[q] * c[q]
                v0 = jnp.where(lane == 0, jnp.sum(p), 0.0)
                v1 = jnp.zeros((L,), jnp.float32)
                for k in range(K):
                    j = e * K + k
                    nv = nvecs[j // L]
                    pn = (nv[j % L] & 1) * D
                    s = t[0] * rows_n[j, pl.ds(pn, L)]
                    for q in range(1, Q):
                        s = s + t[q] * rows_n[j, pl.ds(pn + q * L, L)]
                    col = 1 + k
                    if col < L:
                        v0 = jnp.where(lane == col, jnp.sum(s), v0)
                    else:
                        v1 = jnp.where(lane == col - L, jnp.sum(s), v1)
                acc[pl.ds(e * 2 * L, L)] = v0
                acc[pl.ds(e * 2 * L + L, L)] = v1

            pltpu.sync_copy(
                acc, out_hbm.at[pl.ds((base_w + base) * 2 * L, C * 2 * L)])
            return carry

        lax.fori_loop(0, NCH, chunk_body, 0)

    return sg_kernel


def kernel(target_word, context_word, negative_samples, target_table, context_table):
    B = target_word.shape[0]
    K = negative_samples.shape[1]
    V, D = target_table.shape
    tw = target_word.astype(jnp.int32)
    cw = context_word.astype(jnp.int32)
    neg = negative_samples.astype(jnp.int32).reshape(B * K)
    tt2 = target_table.reshape(V // 2, 2 * D)
    ct2 = context_table.reshape(V // 2, 2 * D)
    sg = _make_sc_kernel(B, K, D, V)
    out = sg(tw, cw, neg, tt2, ct2)
    return out.reshape(B, 2 * L)[:, :1 + K]


# speedup vs baseline: 1.0843x; 1.0558x over previous
import functools

import jax
import jax.numpy as jnp
from jax import lax
from jax.experimental import pallas as pl
from jax.experimental.pallas import tpu as pltpu
from jax.experimental.pallas import tpu_sc as plsc

NC = 2
NS = 16
L = 16
NW = NC * NS
W = 128


def _make_sc_kernel(B, K, D, V):
    BW = B // NW
    C = 16
    NCH = BW // C
    Q = D // L
    CK = C * K
    NSTR = -(-CK // 128)

    mesh = plsc.VectorSubcoreMesh(core_axis_name="c", subcore_axis_name="s")

    @functools.partial(
        pl.kernel,
        out_type=jax.ShapeDtypeStruct((B * 2 * L,), jnp.float32),
        mesh=mesh,
        scratch_types=[
            pltpu.VMEM((BW,), jnp.int32),
            pltpu.VMEM((BW,), jnp.int32),
            pltpu.VMEM((BW * K,), jnp.int32),
            pltpu.VMEM((C, W), jnp.float32),
            pltpu.VMEM((C, W), jnp.float32),
            pltpu.VMEM((CK, W), jnp.float32),
            pltpu.VMEM((C * 2 * L,), jnp.float32),
            pltpu.SemaphoreType.DMA,
        ],
        compiler_params=pltpu.CompilerParams(needs_layout_passes=False),
    )
    def sg_kernel(tw_hbm, cw_hbm, neg_hbm, tt_hbm, ct_hbm, out_hbm,
                  idx_t, idx_c, idx_n, rows_t, rows_c, rows_n, acc, sem):
        wid = lax.axis_index("s") * NC + lax.axis_index("c")
        base_w = wid * BW

        pltpu.sync_copy(tw_hbm.at[pl.ds(base_w, BW)], idx_t)
        pltpu.sync_copy(cw_hbm.at[pl.ds(base_w, BW)], idx_c)
        pltpu.sync_copy(neg_hbm.at[pl.ds(base_w * K, BW * K)], idx_n)

        lane = lax.iota(jnp.int32, L)

        def chunk_body(ci, carry):
            base = ci * C
            copies = [
                pltpu.async_copy(tt_hbm.at[idx_t.at[pl.ds(base, C)]],
                                 rows_t, sem),
                pltpu.async_copy(ct_hbm.at[idx_c.at[pl.ds(base, C)]],
                                 rows_c, sem),
            ]
            for j in range(NSTR):
                n = min(128, CK - j * 128)
                copies.append(pltpu.async_copy(
                    ct_hbm.at[idx_n.at[pl.ds(base * K + j * 128, n)]],
                    rows_n.at[pl.ds(j * 128, n)], sem))
            for cp in copies:
                cp.wait()

            for e in range(C):
                t = [rows_t[e, pl.ds(q * L, L)] for q in range(Q)]
                c = [rows_c[e, pl.ds(q * L, L)] for q in range(Q)]
                p = t[0] * c[0]
                for q in range(1, Q):
                    p = p + t[q] * c[q]
                v0 = jnp.where(lane == 0, jnp.sum(p), 0.0)
                v1 = jnp.zeros((L,), jnp.float32)
                for k in range(K):
                    j = e * K + k
                    s = t[0] * rows_n[j, pl.ds(0, L)]
                    for q in range(1, Q):
                        s = s + t[q] * rows_n[j, pl.ds(q * L, L)]
                    col = 1 + k
                    if col < L:
                        v0 = jnp.where(lane == col, jnp.sum(s), v0)
                    else:
                        v1 = jnp.where(lane == col - L, jnp.sum(s), v1)
                acc[pl.ds(e * 2 * L, L)] = v0
                acc[pl.ds(e * 2 * L + L, L)] = v1

            pltpu.sync_copy(
                acc, out_hbm.at[pl.ds((base_w + base) * 2 * L, C * 2 * L)])
            return carry

        lax.fori_loop(0, NCH, chunk_body, 0)

    return sg_kernel


def kernel(target_word, context_word, negative_samples, target_table, context_table):
    B = target_word.shape[0]
    K = negative_samples.shape[1]
    V, D = target_table.shape
    tw = target_word.astype(jnp.int32)
    cw = context_word.astype(jnp.int32)
    neg = negative_samples.astype(jnp.int32).reshape(B * K)
    ttp = jnp.pad(target_table, ((0, 0), (0, W - D)))
    ctp = jnp.pad(context_table, ((0, 0), (0, W - D)))
    sg = _make_sc_kernel(B, K, D, V)
    out = sg(tw, cw, neg, ttp, ctp)
    return out.reshape(B, 2 * L)[:, :1 + K]


# speedup vs baseline: 1.3376x; 1.2336x over previous
import functools

import jax
import jax.numpy as jnp
from jax import lax
from jax.experimental import pallas as pl
from jax.experimental.pallas import tpu as pltpu
from jax.experimental.pallas import tpu_sc as plsc

NC = 2
NS = 16
L = 16
NW = NC * NS
W = 128


def _make_sc_kernel(B, K, D, V):
    BW = B // NW
    C = 16
    NCH = BW // C
    Q = D // L
    CK = C * K
    NSTR = -(-CK // 128)

    mesh = plsc.VectorSubcoreMesh(core_axis_name="c", subcore_axis_name="s")

    @functools.partial(
        pl.kernel,
        out_type=jax.ShapeDtypeStruct((B * 2 * L,), jnp.float32),
        mesh=mesh,
        scratch_types=[
            pltpu.VMEM((BW,), jnp.int32),
            pltpu.VMEM((BW,), jnp.int32),
            pltpu.VMEM((BW * K,), jnp.int32),
            pltpu.VMEM((C * 8, D), jnp.float32),
            pltpu.VMEM((C, W), jnp.float32),
            pltpu.VMEM((CK, W), jnp.float32),
            pltpu.VMEM((C * 2 * L,), jnp.float32),
            pltpu.SemaphoreType.DMA,
        ],
        compiler_params=pltpu.CompilerParams(needs_layout_passes=False),
    )
    def sg_kernel(tw_hbm, cw_hbm, neg_hbm, tt_hbm, ct_hbm, out_hbm,
                  idx_t, idx_c, idx_n, rows_t, rows_c, rows_n, acc, sem):
        wid = lax.axis_index("s") * NC + lax.axis_index("c")
        base_w = wid * BW

        pltpu.sync_copy(tw_hbm.at[pl.ds(base_w, BW)], idx_t)
        pltpu.sync_copy(cw_hbm.at[pl.ds(base_w, BW)], idx_c)
        pltpu.sync_copy(neg_hbm.at[pl.ds(base_w * K, BW * K)], idx_n)

        lane = lax.iota(jnp.int32, L)

        def chunk_body(ci, carry):
            base = ci * C
            tvec = idx_t[pl.ds(base, L)]
            ta = (tvec >> 3) << 3
            copies = [
                pltpu.async_copy(ct_hbm.at[idx_c.at[pl.ds(base, C)]],
                                 rows_c, sem),
            ]
            for e in range(C):
                copies.append(pltpu.async_copy(
                    tt_hbm.at[pl.ds(pl.multiple_of(ta[e], 8), 8)],
                    rows_t.at[pl.ds(e * 8, 8)], sem))
            for j in range(NSTR):
                n = min(128, CK - j * 128)
                copies.append(pltpu.async_copy(
                    ct_hbm.at[idx_n.at[pl.ds(base * K + j * 128, n)]],
                    rows_n.at[pl.ds(j * 128, n)], sem))
            for cp in copies:
                cp.wait()

            for e in range(C):
                st = tvec[e] & 7
                t = [rows_t[e * 8 + st, pl.ds(q * L, L)] for q in range(Q)]
                c = [rows_c[e, pl.ds(q * L, L)] for q in range(Q)]
                p = t[0] * c[0]
                for q in range(1, Q):
                    p = p + t[q] * c[q]
                v0 = jnp.where(lane == 0, jnp.sum(p), 0.0)
                v1 = jnp.zeros((L,), jnp.float32)
                for k in range(K):
                    j = e * K + k
                    s = t[0] * rows_n[j, pl.ds(0, L)]
                    for q in range(1, Q):
                        s = s + t[q] * rows_n[j, pl.ds(q * L, L)]
                    col = 1 + k
                    if col < L:
                        v0 = jnp.where(lane == col, jnp.sum(s), v0)
                    else:
                        v1 = jnp.where(lane == col - L, jnp.sum(s), v1)
                acc[pl.ds(e * 2 * L, L)] = v0
                acc[pl.ds(e * 2 * L + L, L)] = v1

            pltpu.sync_copy(
                acc, out_hbm.at[pl.ds((base_w + base) * 2 * L, C * 2 * L)])
            return carry

        lax.fori_loop(0, NCH, chunk_body, 0)

    return sg_kernel


def kernel(target_word, context_word, negative_samples, target_table, context_table):
    B = target_word.shape[0]
    K = negative_samples.shape[1]
    V, D = target_table.shape
    tw = target_word.astype(jnp.int32)
    cw = context_word.astype(jnp.int32)
    neg = negative_samples.astype(jnp.int32).reshape(B * K)
    ctp = jnp.pad(context_table, ((0, 0), (0, W - D)))
    sg = _make_sc_kernel(B, K, D, V)
    out = sg(tw, cw, neg, target_table, ctp)
    return out.reshape(B, 2 * L)[:, :1 + K]
